# baseline (speedup 1.0000x reference)
import functools

import jax
import jax.numpy as jnp
from jax import lax
from jax.experimental import pallas as pl
from jax.experimental.pallas import tpu as pltpu
from jax.experimental.pallas import tpu_sc as plsc

F32 = jnp.float32
G = 128
NW = 32
CH = 13


def _dot(a, b):
    return jnp.dot(a, b, preferred_element_type=F32,
                   precision=lax.Precision.DEFAULT)


def _mlp2_body(x_ref, w1_ref, b1_ref, w2_ref, b2_ref, o_ref):
    h = jnp.maximum(_dot(x_ref[...], w1_ref[...]) + b1_ref[...], 0.0)
    o_ref[...] = _dot(h, w2_ref[...]) + b2_ref[...]


def _edge_enc_body(xt_ref, w1_ref, b1_ref, w2_ref, b2_ref, o_ref, scr_ref):
    r = o_ref.shape[0]
    scr_ref[...] = jnp.transpose(xt_ref[...])
    x = jnp.concatenate([scr_ref[pl.Slice(k, r, 8), :] for k in range(8)],
                        axis=1)
    h = jnp.maximum(_dot(x, w1_ref[...]) + b1_ref[...], 0.0)
    o_ref[...] = _dot(h, w2_ref[...]) + b2_ref[...]


def _node_enc_body(x_ref, w1_ref, b1_ref, w2_ref, b2_ref, o_ref):
    h = jnp.maximum(_dot(x_ref[...], w1_ref[...]) + b1_ref[...], 0.0)
    o = _dot(h, w2_ref[...]) + b2_ref[...]
    o_ref[...] = jnp.concatenate(
        [o, jnp.zeros((o.shape[0], o_ref.shape[1] - o.shape[1]), F32)], axis=1)


def _rel1_body(gj_ref, gi_ref, xt_ref, we1_ref, be1_ref, we2_ref, be2_ref,
               wi_ref, wj_ref, wc_ref, b1_ref, w2_ref, b2_ref, w3_ref,
               b3_ref, o_ref, scr_ref):
    r = o_ref.shape[0]
    scr_ref[...] = jnp.transpose(xt_ref[...])
    x = jnp.concatenate([scr_ref[pl.Slice(k, r, 8), :] for k in range(8)],
                        axis=1)
    he = jnp.maximum(_dot(x, we1_ref[...]) + be1_ref[...], 0.0)
    e0 = _dot(he, we2_ref[...]) + be2_ref[...]
    h1 = jnp.maximum(
        _dot(gi_ref[...], wi_ref[...]) + _dot(gj_ref[...], wj_ref[...])
        + _dot(e0, wc_ref[...]) + b1_ref[...], 0.0)
    h2 = jnp.maximum(_dot(h1, w2_ref[...]) + b2_ref[...], 0.0)
    o_ref[...] = _dot(h2, w3_ref[...]) + b3_ref[...]


def _rel_body(gj_ref, gi_ref, e_ref, wi_ref, wj_ref, wc_ref, b1_ref,
              w2_ref, b2_ref, w3_ref, b3_ref, o_ref):
    h1 = jnp.maximum(
        _dot(gi_ref[...], wi_ref[...]) + _dot(gj_ref[...], wj_ref[...])
        + _dot(e_ref[...], wc_ref[...]) + b1_ref[...], 0.0)
    h2 = jnp.maximum(_dot(h1, w2_ref[...]) + b2_ref[...], 0.0)
    o_ref[...] = _dot(h2, w3_ref[...]) + b3_ref[...]


def _obj_body(n_ref, ap_ref, wn_ref, wa_ref, b1_ref, w2_ref, b2_ref,
              w3_ref, b3_ref, o_ref):
    aggr = ap_ref[0] + ap_ref[1]
    h1 = jnp.maximum(
        _dot(n_ref[...], wn_ref[...]) + _dot(aggr, wa_ref[...])
        + b1_ref[...], 0.0)
    h2 = jnp.maximum(_dot(h1, w2_ref[...]) + b2_ref[...], 0.0)
    o_ref[...] = _dot(h2, w3_ref[...]) + b3_ref[...]


def _rel_head_body(gj_ref, gi_ref, e_ref, wi_ref, wj_ref, wc_ref, b1_ref,
                   w2_ref, b2_ref, w3_ref, b3_ref, wh1_ref, bh1_ref,
                   wh2_ref, bh2_ref, o_ref):
    h1 = jnp.maximum(
        _dot(gi_ref[...], wi_ref[...]) + _dot(gj_ref[...], wj_ref[...])
        + _dot(e_ref[...], wc_ref[...]) + b1_ref[...], 0.0)
    h2 = jnp.maximum(_dot(h1, w2_ref[...]) + b2_ref[...], 0.0)
    et = _dot(h2, w3_ref[...]) + b3_ref[...]
    hh = jnp.maximum(_dot(et, wh1_ref[...]) + bh1_ref[...], 0.0)
    z = _dot(hh, wh2_ref[...]) + bh2_ref[...]
    o_ref[...] = 1.0 / (1.0 + jnp.exp(-z))


def _zero_map(ndim, nargs=0):
    return lambda *args: (0,) * ndim


def _off_map(off):
    return lambda i: (i + off, 0)


def _row_call(body, ins, weights, out_cols, rows_per_blk,
              offsets=None, nrows=None, scales=None):
    if nrows is None:
        nrows = ins[0].shape[0]
    grid = nrows // rows_per_blk
    if offsets is None:
        offsets = [0] * len(ins)
    if scales is None:
        scales = [1] * len(ins)
    in_specs = [
        pl.BlockSpec((rows_per_blk * sc, a.shape[1]), _off_map(off))
        for a, off, sc in zip(ins, offsets, scales)
    ]
    in_specs += [pl.BlockSpec(w.shape, _zero_map(w.ndim, 1)) for w in weights]
    return pl.pallas_call(
        body,
        grid=(grid,),
        in_specs=in_specs,
        out_specs=pl.BlockSpec((rows_per_blk, out_cols), lambda i: (i, 0)),
        out_shape=jax.ShapeDtypeStruct((nrows, out_cols), F32),
    )(*ins, *weights)


def _full_call(body, args, out_shape):
    return pl.pallas_call(
        body,
        in_specs=[pl.BlockSpec(a.shape, _zero_map(a.ndim)) for a in args],
        out_specs=pl.BlockSpec(out_shape, _zero_map(len(out_shape))),
        out_shape=jax.ShapeDtypeStruct(out_shape, F32),
    )(*args)


@functools.cache
def _sc_gather_build(n_rows, half_groups, d):
    gpw = half_groups // NW
    rem = half_groups - gpw * NW
    nch = gpw // CH
    assert nch * CH == gpw and nch % 2 == 0
    assert (G * d) % 128 == 0
    GR = G * d // 128
    RPC = CH * GR
    HR = half_groups * GR
    mesh = plsc.VectorSubcoreMesh(core_axis_name="c", subcore_axis_name="s")

    def body(table, idx3, out, ibuf0, ibuf1, rbuf0, rbuf1,
             si0, si1, sg, so0, so1):
        w = lax.axis_index("s") * 2 + lax.axis_index("c")
        ibufs, rbufs = [ibuf0, ibuf1], [rbuf0, rbuf1]
        sis, sos = [si0, si1], [so0, so1]

        for h in (0, 1):
            def idx_src(k, h=h):
                return idx3.at[h, pl.ds(w * gpw + k * CH, CH)]

            def out_dst(k, h=h):
                return out.at[pl.ds((h * half_groups + w * gpw + k * CH) * G,
                                    CH * G)]

            pltpu.async_copy(idx_src(0), ibuf0, si0)
            pltpu.async_copy(idx_src(1), ibuf1, si1)

            def pair(kk, carry):
                for b in range(2):
                    k = 2 * kk + b
                    pltpu.make_async_copy(idx_src(k), ibufs[b], sis[b]).wait()

                    @pl.when(kk >= 1)
                    def _():
                        pltpu.make_async_copy(rbufs[b], out_dst(k - 2),
                                              sos[b]).wait()

                    descs = [
                        pltpu.async_copy(table.at[ibufs[b].at[j]],
                                         rbufs[b].at[pl.ds(j * G, G)], sg)
                        for j in range(CH)
                    ]
                    for dsc in descs:
                        dsc.wait()

                    @pl.when(k + 2 < nch)
                    def _():
                        pltpu.async_copy(idx_src(k + 2), ibufs[b], sis[b])

                    pltpu.async_copy(rbufs[b], out_dst(k), sos[b])
                return carry

            lax.fori_loop(0, nch // 2, pair, 0)
            pltpu.make_async_copy(rbuf0, out_dst(nch - 2), so0).wait()
            pltpu.make_async_copy(rbuf1, out_dst(nch - 1), so1).wait()

            if rem:
                @pl.when(w < rem)
                def _(h=h):
                    tg = gpw * NW + w
                    pltpu.sync_copy(idx3.at[h, pl.ds(tg, 1)],
                                    ibuf0.at[pl.ds(0, 1)])
                    pltpu.async_copy(table.at[ibuf0.at[0]],
                                     rbuf0.at[pl.ds(0, G)], sg).wait()
                    pltpu.sync_copy(
                        rbuf0.at[pl.ds(0, G)],
                        out.at[pl.ds((h * half_groups + tg) * G, G)])

    return pl.kernel(
        body,
        out_type=jax.ShapeDtypeStruct((2 * half_groups * G, d), F32),
        mesh=mesh,
        compiler_params=pltpu.CompilerParams(use_tc_tiling_on_sc=False),
        scratch_types=[
            pltpu.VMEM((CH, G), jnp.int32),
            pltpu.VMEM((CH, G), jnp.int32),
            pltpu.VMEM((CH * G, d), F32),
            pltpu.VMEM((CH * G, d), F32),
            pltpu.SemaphoreType.DMA,
            pltpu.SemaphoreType.DMA,
            pltpu.SemaphoreType.DMA,
            pltpu.SemaphoreType.DMA,
            pltpu.SemaphoreType.DMA,
        ],
    )


@functools.cache
def _sc_scatter_build(n_rows, total_groups, d):
    SGR = 8
    n_sg = total_groups // SGR
    rem_rows = total_groups - n_sg * SGR
    base_sg, extra = divmod(n_sg, NW)
    dump_a = (n_rows // 16) & ~7
    dump_b = n_rows - 15 * dump_a
    mesh = plsc.VectorSubcoreMesh(core_axis_name="c", subcore_axis_name="s")

    def body(e, idx3, z, out, ibuf0, ibuf1, ebuf0, ebuf1, acc,
             si0, si1, se0, se1, sa):
        c = lax.axis_index("c")
        s = lax.axis_index("s")
        w = s * 2 + c
        ibufs, ebufs = [ibuf0, ibuf1], [ebuf0, ebuf1]
        sis, ses = [si0, si1], [se0, se1]
        nsg = base_sg + jnp.where(w < extra, 1, 0)

        @pl.when(s < 15)
        def _():
            pltpu.sync_copy(z.at[pl.ds(0, dump_a)], acc.at[pl.ds(s * dump_a, dump_a)])

        @pl.when(s == 15)
        def _():
            pltpu.sync_copy(z, acc.at[pl.ds(15 * dump_a, dump_b)])

        plsc.subcore_barrier()

        def idx_src(k):
            return idx3.at[1, pl.ds((k * NW + w) * SGR, SGR)]

        def e_src(k):
            return e.at[pl.ds((k * NW + w) * SGR * G, SGR * G)]

        def prefetch(k, b):
            @pl.when(k < nsg)
            def _():
                pltpu.async_copy(idx_src(k), ibufs[b], sis[b])
                pltpu.async_copy(e_src(k), ebufs[b], ses[b])

        prefetch(0, 0)
        prefetch(1, 1)

        def pair(kk, carry):
            for b in range(2):
                k = 2 * kk + b

                @pl.when(k < nsg)
                def _():
                    pltpu.make_async_copy(idx_src(k), ibufs[b], sis[b]).wait()
                    pltpu.make_async_copy(e_src(k), ebufs[b], ses[b]).wait()
                    descs = [
                        pltpu.async_copy(ebufs[b].at[pl.ds(j * G, G)],
                                         acc.at[ibufs[b].at[j]], sa, add=True)
                        for j in range(SGR)
                    ]
                    for dsc in descs:
                        dsc.wait()
                    prefetch(k + 2, b)
            return carry

        lax.fori_loop(0, (base_sg + 1 + 1) // 2, pair, 0)

        if rem_rows:
            @pl.when(w == 0)
            def _():
                bg = n_sg * SGR
                pltpu.sync_copy(idx3.at[1, pl.ds(bg, rem_rows)],
                                ibuf0.at[pl.ds(0, rem_rows)])
                pltpu.sync_copy(e.at[pl.ds(bg * G, rem_rows * G)],
                                ebuf0.at[pl.ds(0, rem_rows * G)])
                for j in range(rem_rows):
                    pltpu.sync_copy(ebuf0.at[pl.ds(j * G, G)],
                                    acc.at[ibuf0.at[j]], add=True)

        plsc.subcore_barrier()

        @pl.when(s < 15)
        def _():
            pltpu.sync_copy(acc.at[pl.ds(s * dump_a, dump_a)],
                            out.at[pl.ds(c * n_rows + s * dump_a, dump_a)])

        @pl.when(s == 15)
        def _():
            pltpu.sync_copy(acc.at[pl.ds(15 * dump_a, dump_b)],
                            out.at[pl.ds(c * n_rows + 15 * dump_a, dump_b)])

    return pl.kernel(
        body,
        out_type=jax.ShapeDtypeStruct((2 * n_rows, d), F32),
        mesh=mesh,
        compiler_params=pltpu.CompilerParams(use_tc_tiling_on_sc=False),
        scratch_types=[
            pltpu.VMEM((SGR, G), jnp.int32),
            pltpu.VMEM((SGR, G), jnp.int32),
            pltpu.VMEM((SGR * G, d), F32),
            pltpu.VMEM((SGR * G, d), F32),
            pltpu.VMEM_SHARED((n_rows, d), F32),
            pltpu.SemaphoreType.DMA,
            pltpu.SemaphoreType.DMA,
            pltpu.SemaphoreType.DMA,
            pltpu.SemaphoreType.DMA,
            pltpu.SemaphoreType.DMA,
        ],
    )


def _sc_gather(table, idx3):
    k = _sc_gather_build(table.shape[0], idx3.shape[1], table.shape[1])
    return k(table, idx3)


def _sc_scatter(e2, idx3, z, n_rows):
    k = _sc_scatter_build(n_rows, idx3.shape[1], e2.shape[1])
    return k(e2, idx3, z)


def _kron8(w):
    return jnp.kron(jnp.eye(8, dtype=F32), w)


def _padr(w, rows):
    return jnp.concatenate([w, jnp.zeros((rows - w.shape[0], w.shape[1]), F32)], 0)


def _padc(w, cols):
    return jnp.concatenate([w, jnp.zeros((w.shape[0], cols - w.shape[1]), F32)], 1)


def _tile8(b):
    return jnp.tile(b, 8)[None, :]


def kernel(x, edge_index, edge_attr, params):
    N = x.shape[0]
    E = edge_index.shape[1]
    node_lat = 8
    nlat_p = 16
    elat_p = 16

    (wn1, bn1), (wn2, bn2) = params["node_enc"]
    node = _full_call(_node_enc_body,
                      (x, wn1, bn1[None, :], wn2, bn2[None, :]),
                      (N, nlat_p))

    (we1, be1), (we2, be2) = params["edge_enc"]
    ew = [_kron8(we1), _tile8(be1),
          _kron8(_padc(we2, elat_p)), _tile8(_padc(be2[None, :], elat_p)[0])]
    edge = None

    eidx3 = edge_index.reshape(2, E // G, G)
    z = jnp.zeros((N - 15 * ((N // 16) & ~7), elat_p), F32)

    n_layers = len(params["gnn"])
    RB = 2000
    HB = (E * nlat_p // 128) // RB
    for li, (rel, obj) in enumerate(params["gnn"]):
        (wr1, br1), (wr2, br2), (wr3, br3) = rel
        gath = _sc_gather(node.reshape(N, nlat_p), eidx3)
        gath = gath.reshape(2 * E * nlat_p // 128, 128)
        wi = _kron8(_padr(wr1[:node_lat], nlat_p))
        wj = _kron8(_padr(wr1[node_lat:2 * node_lat], nlat_p))
        wc = _kron8(_padr(wr1[2 * node_lat:], elat_p))
        rw = [wi, wj, wc, _tile8(br1),
              _kron8(wr2), _tile8(br2),
              _kron8(_padc(wr3, elat_p)),
              _tile8(_padc(br3[None, :], elat_p)[0])]
        if li == n_layers - 1:
            (wh1, bh1), (wh2, bh2) = params["w_head"]
            rw += [_kron8(_padr(wh1, elat_p)), _tile8(bh1),
                   _kron8(wh2), _tile8(bh2)]
            out = pl.pallas_call(
                _rel_head_body,
                grid=(E // 8 // RB,),
                in_specs=[pl.BlockSpec((RB, 128), _off_map(0)),
                          pl.BlockSpec((RB, 128), _off_map(HB)),
                          pl.BlockSpec((RB, 128), _off_map(0))]
                + [pl.BlockSpec(w.shape, _zero_map(w.ndim, 1)) for w in rw],
                out_specs=pl.BlockSpec((RB, 8), lambda i: (i, 0)),
                out_shape=jax.ShapeDtypeStruct((E // 8, 8), F32),
            )(gath, gath, edge, *rw)
            return out.reshape(E, 1, 1)
        if li == 0:
            et = pl.pallas_call(
                _rel1_body,
                grid=(E // 8 // RB,),
                in_specs=[pl.BlockSpec((RB, 128), _off_map(0)),
                          pl.BlockSpec((RB, 128), _off_map(HB)),
                          pl.BlockSpec((edge_attr.shape[1], 8 * RB),
                                       lambda i: (0, i))]
                + [pl.BlockSpec(w.shape, _zero_map(w.ndim, 1))
                   for w in ew + rw],
                out_specs=pl.BlockSpec((RB, 8 * elat_p), lambda i: (i, 0)),
                out_shape=jax.ShapeDtypeStruct((E // 8, 8 * elat_p), F32),
                scratch_shapes=[pltpu.VMEM((8 * RB, edge_attr.shape[1]), F32)],
            )(gath, gath, edge_attr.T, *ew, *rw)
        else:
            et = _row_call(_rel_body, [gath, gath, edge], rw,
                           8 * elat_p, RB,
                           offsets=[0, HB, 0],
                           nrows=E // 8)
        if li < n_layers - 1:
            (wo1, bo1), (wo2, bo2), (wo3, bo3) = obj
            ap = _sc_scatter(et.reshape(E, elat_p), eidx3, z, N)
            ap = ap.reshape(2, N // 8, 8 * elat_p)
            wn = _kron8(_padr(wo1[:node_lat], nlat_p))
            wa = _kron8(_padr(wo1[node_lat:], elat_p))
            node = _full_call(
                _obj_body,
                (node.reshape(N // 8, 8 * nlat_p), ap,
                 wn, wa, _tile8(bo1),
                 _kron8(wo2), _tile8(bo2),
                 _kron8(_padc(wo3, nlat_p)), _tile8(_padc(bo3[None, :], nlat_p)[0])),
                (N // 8, 8 * nlat_p))
        edge = et


# speedup vs baseline: 1.0203x; 1.0203x over previous
import functools

import jax
import jax.numpy as jnp
from jax import lax
from jax.experimental import pallas as pl
from jax.experimental.pallas import tpu as pltpu
from jax.experimental.pallas import tpu_sc as plsc

F32 = jnp.float32
G = 128
NW = 32
CH = 13


def _dot(a, b):
    return jnp.dot(a, b, preferred_element_type=F32,
                   precision=lax.Precision.DEFAULT)


def _mlp2_body(x_ref, w1_ref, b1_ref, w2_ref, b2_ref, o_ref):
    h = jnp.maximum(_dot(x_ref[...], w1_ref[...]) + b1_ref[...], 0.0)
    o_ref[...] = _dot(h, w2_ref[...]) + b2_ref[...]


def _edge_enc_body(xt_ref, w1_ref, b1_ref, w2_ref, b2_ref, o_ref, scr_ref):
    r = o_ref.shape[0]
    scr_ref[...] = jnp.transpose(xt_ref[...])
    x = jnp.concatenate([scr_ref[pl.Slice(k, r, 8), :] for k in range(8)],
                        axis=1)
    h = jnp.maximum(_dot(x, w1_ref[...]) + b1_ref[...], 0.0)
    o_ref[...] = _dot(h, w2_ref[...]) + b2_ref[...]


def _node_enc_body(x_ref, w1_ref, b1_ref, w2_ref, b2_ref, o_ref):
    h = jnp.maximum(_dot(x_ref[...], w1_ref[...]) + b1_ref[...], 0.0)
    o = _dot(h, w2_ref[...]) + b2_ref[...]
    o_ref[...] = jnp.concatenate(
        [o, jnp.zeros((o.shape[0], o_ref.shape[1] - o.shape[1]), F32)], axis=1)


def _rel_body(gj_ref, gi_ref, e_ref, wi_ref, wj_ref, wc_ref, b1_ref,
              w2_ref, b2_ref, w3_ref, b3_ref, o_ref):
    h1 = jnp.maximum(
        _dot(gi_ref[...], wi_ref[...]) + _dot(gj_ref[...], wj_ref[...])
        + _dot(e_ref[...], wc_ref[...]) + b1_ref[...], 0.0)
    h2 = jnp.maximum(_dot(h1, w2_ref[...]) + b2_ref[...], 0.0)
    o_ref[...] = _dot(h2, w3_ref[...]) + b3_ref[...]


def _obj_body(n_ref, ap_ref, wn_ref, wa_ref, b1_ref, w2_ref, b2_ref,
              w3_ref, b3_ref, o_ref):
    aggr = ap_ref[0] + ap_ref[1]
    h1 = jnp.maximum(
        _dot(n_ref[...], wn_ref[...]) + _dot(aggr, wa_ref[...])
        + b1_ref[...], 0.0)
    h2 = jnp.maximum(_dot(h1, w2_ref[...]) + b2_ref[...], 0.0)
    o_ref[...] = _dot(h2, w3_ref[...]) + b3_ref[...]


def _rel_head_body(gj_ref, gi_ref, e_ref, wi_ref, wj_ref, wc_ref, b1_ref,
                   w2_ref, b2_ref, w3_ref, b3_ref, wh1_ref, bh1_ref,
                   wh2_ref, bh2_ref, o_ref):
    h1 = jnp.maximum(
        _dot(gi_ref[...], wi_ref[...]) + _dot(gj_ref[...], wj_ref[...])
        + _dot(e_ref[...], wc_ref[...]) + b1_ref[...], 0.0)
    h2 = jnp.maximum(_dot(h1, w2_ref[...]) + b2_ref[...], 0.0)
    et = _dot(h2, w3_ref[...]) + b3_ref[...]
    hh = jnp.maximum(_dot(et, wh1_ref[...]) + bh1_ref[...], 0.0)
    z = _dot(hh, wh2_ref[...]) + bh2_ref[...]
    o_ref[...] = 1.0 / (1.0 + jnp.exp(-z))


def _zero_map(ndim, nargs=0):
    return lambda *args: (0,) * ndim


def _off_map(off):
    return lambda i: (i + off, 0)


def _row_call(body, ins, weights, out_cols, rows_per_blk,
              offsets=None, nrows=None, scales=None):
    if nrows is None:
        nrows = ins[0].shape[0]
    grid = nrows // rows_per_blk
    if offsets is None:
        offsets = [0] * len(ins)
    if scales is None:
        scales = [1] * len(ins)
    in_specs = [
        pl.BlockSpec((rows_per_blk * sc, a.shape[1]), _off_map(off))
        for a, off, sc in zip(ins, offsets, scales)
    ]
    in_specs += [pl.BlockSpec(w.shape, _zero_map(w.ndim, 1)) for w in weights]
    return pl.pallas_call(
        body,
        grid=(grid,),
        in_specs=in_specs,
        out_specs=pl.BlockSpec((rows_per_blk, out_cols), lambda i: (i, 0)),
        out_shape=jax.ShapeDtypeStruct((nrows, out_cols), F32),
    )(*ins, *weights)


def _full_call(body, args, out_shape):
    return pl.pallas_call(
        body,
        in_specs=[pl.BlockSpec(a.shape, _zero_map(a.ndim)) for a in args],
        out_specs=pl.BlockSpec(out_shape, _zero_map(len(out_shape))),
        out_shape=jax.ShapeDtypeStruct(out_shape, F32),
    )(*args)


@functools.cache
def _sc_gather_build(n_rows, half_groups, d):
    gpw = half_groups // NW
    rem = half_groups - gpw * NW
    nch = gpw // CH
    assert nch * CH == gpw and nch % 2 == 0
    assert (G * d) % 128 == 0
    GR = G * d // 128
    RPC = CH * GR
    HR = half_groups * GR
    mesh = plsc.VectorSubcoreMesh(core_axis_name="c", subcore_axis_name="s")

    def body(table, idx3, out, ibuf0, ibuf1, rbuf0, rbuf1,
             si0, si1, sg, so0, so1):
        w = lax.axis_index("s") * 2 + lax.axis_index("c")
        ibufs, rbufs = [ibuf0, ibuf1], [rbuf0, rbuf1]
        sis, sos = [si0, si1], [so0, so1]

        for h in (0, 1):
            def idx_src(k, h=h):
                return idx3.at[h, pl.ds(w * gpw + k * CH, CH)]

            def out_dst(k, h=h):
                return out.at[pl.ds((h * half_groups + w * gpw + k * CH) * G,
                                    CH * G)]

            pltpu.async_copy(idx_src(0), ibuf0, si0)
            pltpu.async_copy(idx_src(1), ibuf1, si1)

            def pair(kk, carry):
                for b in range(2):
                    k = 2 * kk + b
                    pltpu.make_async_copy(idx_src(k), ibufs[b], sis[b]).wait()

                    @pl.when(kk >= 1)
                    def _():
                        pltpu.make_async_copy(rbufs[b], out_dst(k - 2),
                                              sos[b]).wait()

                    descs = [
                        pltpu.async_copy(table.at[ibufs[b].at[j]],
                                         rbufs[b].at[pl.ds(j * G, G)], sg)
                        for j in range(CH)
                    ]
                    for dsc in descs:
                        dsc.wait()

                    @pl.when(k + 2 < nch)
                    def _():
                        pltpu.async_copy(idx_src(k + 2), ibufs[b], sis[b])

                    pltpu.async_copy(rbufs[b], out_dst(k), sos[b])
                return carry

            lax.fori_loop(0, nch // 2, pair, 0)
            pltpu.make_async_copy(rbuf0, out_dst(nch - 2), so0).wait()
            pltpu.make_async_copy(rbuf1, out_dst(nch - 1), so1).wait()

            if rem:
                @pl.when(w < rem)
                def _(h=h):
                    tg = gpw * NW + w
                    pltpu.sync_copy(idx3.at[h, pl.ds(tg, 1)],
                                    ibuf0.at[pl.ds(0, 1)])
                    pltpu.async_copy(table.at[ibuf0.at[0]],
                                     rbuf0.at[pl.ds(0, G)], sg).wait()
                    pltpu.sync_copy(
                        rbuf0.at[pl.ds(0, G)],
                        out.at[pl.ds((h * half_groups + tg) * G, G)])

    return pl.kernel(
        body,
        out_type=jax.ShapeDtypeStruct((2 * half_groups * G, d), F32),
        mesh=mesh,
        compiler_params=pltpu.CompilerParams(use_tc_tiling_on_sc=False),
        scratch_types=[
            pltpu.VMEM((CH, G), jnp.int32),
            pltpu.VMEM((CH, G), jnp.int32),
            pltpu.VMEM((CH * G, d), F32),
            pltpu.VMEM((CH * G, d), F32),
            pltpu.SemaphoreType.DMA,
            pltpu.SemaphoreType.DMA,
            pltpu.SemaphoreType.DMA,
            pltpu.SemaphoreType.DMA,
            pltpu.SemaphoreType.DMA,
        ],
    )


@functools.cache
def _sc_scatter_build(n_rows, total_groups, d):
    SGR = 8
    n_sg = total_groups // SGR
    rem_rows = total_groups - n_sg * SGR
    base_sg, extra = divmod(n_sg, NW)
    dump_a = (n_rows // 16) & ~7
    dump_b = n_rows - 15 * dump_a
    mesh = plsc.VectorSubcoreMesh(core_axis_name="c", subcore_axis_name="s")

    def body(e, idx3, z, out, ibuf0, ibuf1, ebuf0, ebuf1, acc,
             si0, si1, se0, se1, sa):
        c = lax.axis_index("c")
        s = lax.axis_index("s")
        w = s * 2 + c
        ibufs, ebufs = [ibuf0, ibuf1], [ebuf0, ebuf1]
        sis, ses = [si0, si1], [se0, se1]
        nsg = base_sg + jnp.where(w < extra, 1, 0)

        @pl.when(s < 15)
        def _():
            pltpu.sync_copy(z.at[pl.ds(0, dump_a)], acc.at[pl.ds(s * dump_a, dump_a)])

        @pl.when(s == 15)
        def _():
            pltpu.sync_copy(z, acc.at[pl.ds(15 * dump_a, dump_b)])

        plsc.subcore_barrier()

        def idx_src(k):
            return idx3.at[1, pl.ds((k * NW + w) * SGR, SGR)]

        def e_src(k):
            return e.at[pl.ds((k * NW + w) * SGR * G, SGR * G)]

        def prefetch(k, b):
            @pl.when(k < nsg)
            def _():
                pltpu.async_copy(idx_src(k), ibufs[b], sis[b])
                pltpu.async_copy(e_src(k), ebufs[b], ses[b])

        prefetch(0, 0)
        prefetch(1, 1)

        def pair(kk, carry):
            for b in range(2):
                k = 2 * kk + b

                @pl.when(k < nsg)
                def _():
                    pltpu.make_async_copy(idx_src(k), ibufs[b], sis[b]).wait()
                    pltpu.make_async_copy(e_src(k), ebufs[b], ses[b]).wait()
                    descs = [
                        pltpu.async_copy(ebufs[b].at[pl.ds(j * G, G)],
                                         acc.at[ibufs[b].at[j]], sa, add=True)
                        for j in range(SGR)
                    ]
                    for dsc in descs:
                        dsc.wait()
                    prefetch(k + 2, b)
            return carry

        lax.fori_loop(0, (base_sg + 1 + 1) // 2, pair, 0)

        if rem_rows:
            @pl.when(w == 0)
            def _():
                bg = n_sg * SGR
                pltpu.sync_copy(idx3.at[1, pl.ds(bg, rem_rows)],
                                ibuf0.at[pl.ds(0, rem_rows)])
                pltpu.sync_copy(e.at[pl.ds(bg * G, rem_rows * G)],
                                ebuf0.at[pl.ds(0, rem_rows * G)])
                for j in range(rem_rows):
                    pltpu.sync_copy(ebuf0.at[pl.ds(j * G, G)],
                                    acc.at[ibuf0.at[j]], add=True)

        plsc.subcore_barrier()

        @pl.when(s < 15)
        def _():
            pltpu.sync_copy(acc.at[pl.ds(s * dump_a, dump_a)],
                            out.at[pl.ds(c * n_rows + s * dump_a, dump_a)])

        @pl.when(s == 15)
        def _():
            pltpu.sync_copy(acc.at[pl.ds(15 * dump_a, dump_b)],
                            out.at[pl.ds(c * n_rows + 15 * dump_a, dump_b)])

    return pl.kernel(
        body,
        out_type=jax.ShapeDtypeStruct((2 * n_rows, d), F32),
        mesh=mesh,
        compiler_params=pltpu.CompilerParams(use_tc_tiling_on_sc=False),
        scratch_types=[
            pltpu.VMEM((SGR, G), jnp.int32),
            pltpu.VMEM((SGR, G), jnp.int32),
            pltpu.VMEM((SGR * G, d), F32),
            pltpu.VMEM((SGR * G, d), F32),
            pltpu.VMEM_SHARED((n_rows, d), F32),
            pltpu.SemaphoreType.DMA,
            pltpu.SemaphoreType.DMA,
            pltpu.SemaphoreType.DMA,
            pltpu.SemaphoreType.DMA,
            pltpu.SemaphoreType.DMA,
        ],
    )


def _sc_gather(table, idx3):
    k = _sc_gather_build(table.shape[0], idx3.shape[1], table.shape[1])
    return k(table, idx3)


def _sc_scatter(e2, idx3, z, n_rows):
    k = _sc_scatter_build(n_rows, idx3.shape[1], e2.shape[1])
    return k(e2, idx3, z)


def _kron8(w):
    return jnp.kron(jnp.eye(8, dtype=F32), w)


def _padr(w, rows):
    return jnp.concatenate([w, jnp.zeros((rows - w.shape[0], w.shape[1]), F32)], 0)


def _padc(w, cols):
    return jnp.concatenate([w, jnp.zeros((w.shape[0], cols - w.shape[1]), F32)], 1)


def _tile8(b):
    return jnp.tile(b, 8)[None, :]


def kernel(x, edge_index, edge_attr, params):
    N = x.shape[0]
    E = edge_index.shape[1]
    node_lat = 8
    nlat_p = 16
    elat_p = 16

    (wn1, bn1), (wn2, bn2) = params["node_enc"]
    node = _full_call(_node_enc_body,
                      (x, wn1, bn1[None, :], wn2, bn2[None, :]),
                      (N, nlat_p))

    (we1, be1), (we2, be2) = params["edge_enc"]
    ew = [_kron8(we1), _tile8(be1),
          _kron8(_padc(we2, elat_p)), _tile8(_padc(be2[None, :], elat_p)[0])]
    RB0 = 2000
    edge = pl.pallas_call(
        _edge_enc_body,
        grid=(E // 8 // RB0,),
        in_specs=[pl.BlockSpec((edge_attr.shape[1], 8 * RB0),
                               lambda i: (0, i))]
        + [pl.BlockSpec(w.shape, _zero_map(w.ndim, 1)) for w in ew],
        out_specs=pl.BlockSpec((RB0, 8 * elat_p), lambda i: (i, 0)),
        out_shape=jax.ShapeDtypeStruct((E // 8, 8 * elat_p), F32),
        scratch_shapes=[pltpu.VMEM((8 * RB0, edge_attr.shape[1]), F32)],
    )(edge_attr.T, *ew)

    eidx3 = edge_index.reshape(2, E // G, G)
    z = jnp.zeros((N - 15 * ((N // 16) & ~7), elat_p), F32)

    n_layers = len(params["gnn"])
    RB = 2000
    HB = (E * nlat_p // 128) // RB
    for li, (rel, obj) in enumerate(params["gnn"]):
        (wr1, br1), (wr2, br2), (wr3, br3) = rel
        gath = _sc_gather(node.reshape(N, nlat_p), eidx3)
        gath = gath.reshape(2 * E * nlat_p // 128, 128)
        wi = _kron8(_padr(wr1[:node_lat], nlat_p))
        wj = _kron8(_padr(wr1[node_lat:2 * node_lat], nlat_p))
        wc = _kron8(_padr(wr1[2 * node_lat:], elat_p))
        rw = [wi, wj, wc, _tile8(br1),
              _kron8(wr2), _tile8(br2),
              _kron8(_padc(wr3, elat_p)),
              _tile8(_padc(br3[None, :], elat_p)[0])]
        if li == n_layers - 1:
            (wh1, bh1), (wh2, bh2) = params["w_head"]
            rw += [_kron8(_padr(wh1, elat_p)), _tile8(bh1),
                   _kron8(wh2), _tile8(bh2)]
            out = pl.pallas_call(
                _rel_head_body,
                grid=(E // 8 // RB,),
                in_specs=[pl.BlockSpec((RB, 128), _off_map(0)),
                          pl.BlockSpec((RB, 128), _off_map(HB)),
                          pl.BlockSpec((RB, 128), _off_map(0))]
                + [pl.BlockSpec(w.shape, _zero_map(w.ndim, 1)) for w in rw],
                out_specs=pl.BlockSpec((RB, 8), lambda i: (i, 0)),
                out_shape=jax.ShapeDtypeStruct((E // 8, 8), F32),
            )(gath, gath, edge, *rw)
            return out.reshape(E, 1, 1)
        et = _row_call(_rel_body, [gath, gath, edge], rw,
                       8 * elat_p, RB,
                       offsets=[0, HB, 0],
                       nrows=E // 8)
        if li < n_layers - 1:
            (wo1, bo1), (wo2, bo2), (wo3, bo3) = obj
            ap = _sc_scatter(et.reshape(E, elat_p), eidx3, z, N)
            ap = ap.reshape(2, N // 8, 8 * elat_p)
            wn = _kron8(_padr(wo1[:node_lat], nlat_p))
            wa = _kron8(_padr(wo1[node_lat:], elat_p))
            node = _full_call(
                _obj_body,
                (node.reshape(N // 8, 8 * nlat_p), ap,
                 wn, wa, _tile8(bo1),
                 _kron8(wo2), _tile8(bo2),
                 _kron8(_padc(wo3, nlat_p)), _tile8(_padc(bo3[None, :], nlat_p)[0])),
                (N // 8, 8 * nlat_p))
        edge = et
